# baseline (speedup 1.0000x reference)
import functools

import jax
import jax.numpy as jnp
from jax.experimental import pallas as pl
from jax.experimental.pallas import tpu as pltpu

D_MODEL = 768
N_EXPERTS = 64
BT = 4096
KB = 2
KS = D_MODEL // KB


def _router_body(x_ref, w_ref, gates_ref, idx_ref, loss_ref,
                 acc_ref, counts_ref):
    i = pl.program_id(0)
    k = pl.program_id(1)
    nblk = pl.num_programs(0)

    x = x_ref[...]
    w = w_ref[...]
    part = jax.lax.dot_general(w, x, (((1,), (1,)), ((), ())),
                               preferred_element_type=jnp.float32)

    @pl.when(k == 0)
    def _first():
        acc_ref[...] = part

    @pl.when(k == KB - 1)
    def _last():
        lt = acc_ref[...] + part

        row = jax.lax.broadcasted_iota(
            jnp.int32, lt.shape, 0).astype(jnp.float32)
        m1 = jnp.max(lt, axis=0, keepdims=True)
        e1 = jnp.min(jnp.where(lt == m1, row, float(N_EXPERTS)),
                     axis=0, keepdims=True)
        lt2 = jnp.where(row == e1, -jnp.inf, lt)
        m2 = jnp.max(lt2, axis=0, keepdims=True)
        e2 = jnp.min(jnp.where(lt2 == m2, row, float(N_EXPERTS)),
                     axis=0, keepdims=True)

        ed = jnp.exp(m2 - m1)
        s1 = 1.0 / (1.0 + ed)
        s2 = ed * s1

        gates_t = (jnp.where(row == e1, s1, 0.0)
                   + jnp.where(row == e2, s2, 0.0))
        gates_ref[...] = gates_t[None]

        idx_t = jnp.concatenate([e1, e2], axis=0).astype(jnp.int32)
        idx_ref[...] = idx_t[None]

        @pl.when(i == 0)
        def _init():
            counts_ref[...] = jnp.zeros_like(counts_ref)

        counts_ref[...] += jnp.sum(gates_t, axis=1, keepdims=True)

        @pl.when(i == nblk - 1)
        def _finish():
            counts = counts_ref[...]
            total = jnp.sum(counts)
            dev = counts / total * N_EXPERTS - 1.0
            loss_ref[...] = jnp.mean(dev * dev, axis=0, keepdims=True)


@functools.partial(jax.jit, static_argnums=())
def kernel(x, W, n_active, capacity):
    b, s, d = x.shape
    t = b * s
    blk_per_batch = s // BT
    xf = x.reshape(t, d)
    grid = (t // BT, KB)
    gates3, idx3, loss2d = pl.pallas_call(
        _router_body,
        grid=grid,
        in_specs=[
            pl.BlockSpec((BT, KS), lambda i, k: (i, k)),
            pl.BlockSpec((N_EXPERTS, KS), lambda i, k: (0, k)),
        ],
        out_specs=[
            pl.BlockSpec((1, N_EXPERTS, BT),
                         lambda i, k: (i // blk_per_batch, 0,
                                       i % blk_per_batch)),
            pl.BlockSpec((1, 2, BT),
                         lambda i, k: (i // blk_per_batch, 0,
                                       i % blk_per_batch)),
            pl.BlockSpec((1, 1), lambda i, k: (0, 0)),
        ],
        out_shape=[
            jax.ShapeDtypeStruct((b, N_EXPERTS, s), jnp.float32),
            jax.ShapeDtypeStruct((b, 2, s), jnp.int32),
            jax.ShapeDtypeStruct((1, 1), jnp.float32),
        ],
        scratch_shapes=[
            pltpu.VMEM((N_EXPERTS, BT), jnp.float32),
            pltpu.VMEM((N_EXPERTS, 1), jnp.float32),
        ],
    )(xf, W)
    gates = jnp.transpose(gates3, (0, 2, 1))
    idx = jnp.transpose(idx3, (0, 2, 1))
    return gates, idx, loss2d[0, 0]


# speedup vs baseline: 1.2388x; 1.2388x over previous
import functools

import jax
import jax.numpy as jnp
from jax.experimental import pallas as pl
from jax.experimental.pallas import tpu as pltpu

D_MODEL = 768
N_EXPERTS = 64
BT = 4096


def _router_body(x_ref, w_ref, gates_ref, idx_ref, loss_ref, counts_ref):
    i = pl.program_id(0)
    nblk = pl.num_programs(0)

    x = x_ref[...]
    w = w_ref[...]
    lt = jax.lax.dot_general(w, x, (((1,), (1,)), ((), ())),
                             preferred_element_type=jnp.float32)

    row = jax.lax.broadcasted_iota(jnp.int32, lt.shape, 0).astype(jnp.float32)
    m1 = jnp.max(lt, axis=0, keepdims=True)
    e1 = jnp.min(jnp.where(lt == m1, row, float(N_EXPERTS)),
                 axis=0, keepdims=True)
    lt2 = jnp.where(row == e1, -jnp.inf, lt)
    m2 = jnp.max(lt2, axis=0, keepdims=True)
    e2 = jnp.min(jnp.where(lt2 == m2, row, float(N_EXPERTS)),
                 axis=0, keepdims=True)

    ed = jnp.exp(m2 - m1)
    s1 = 1.0 / (1.0 + ed)
    s2 = ed * s1

    gates_t = (jnp.where(row == e1, s1, 0.0)
               + jnp.where(row == e2, s2, 0.0))
    gates_ref[...] = gates_t[None]

    idx_t = jnp.concatenate([e1, e2], axis=0).astype(jnp.int32)
    idx_ref[...] = idx_t[None]

    @pl.when(i == 0)
    def _init():
        counts_ref[...] = jnp.zeros_like(counts_ref)

    counts_ref[...] += jnp.sum(gates_t, axis=1, keepdims=True)

    @pl.when(i == nblk - 1)
    def _finish():
        counts = counts_ref[...]
        total = jnp.sum(counts)
        dev = counts / total * N_EXPERTS - 1.0
        loss_ref[...] = jnp.mean(dev * dev, axis=0, keepdims=True)


@functools.partial(jax.jit, static_argnums=())
def kernel(x, W, n_active, capacity):
    b, s, d = x.shape
    t = b * s
    blk_per_batch = s // BT
    xf = x.reshape(t, d)
    grid = (t // BT,)
    gates3, idx3, loss2d = pl.pallas_call(
        _router_body,
        grid=grid,
        in_specs=[
            pl.BlockSpec((BT, D_MODEL), lambda i: (i, 0)),
            pl.BlockSpec((N_EXPERTS, D_MODEL), lambda i: (0, 0)),
        ],
        out_specs=[
            pl.BlockSpec((1, N_EXPERTS, BT),
                         lambda i: (i // blk_per_batch, 0, i % blk_per_batch)),
            pl.BlockSpec((1, 2, BT),
                         lambda i: (i // blk_per_batch, 0, i % blk_per_batch)),
            pl.BlockSpec((1, 1), lambda i: (0, 0)),
        ],
        out_shape=[
            jax.ShapeDtypeStruct((b, N_EXPERTS, s), jnp.float32),
            jax.ShapeDtypeStruct((b, 2, s), jnp.int32),
            jax.ShapeDtypeStruct((1, 1), jnp.float32),
        ],
        scratch_shapes=[pltpu.VMEM((N_EXPERTS, 1), jnp.float32)],
    )(xf, W)
    gates = jnp.transpose(gates3, (0, 2, 1))
    idx = jnp.transpose(idx3, (0, 2, 1))
    return gates, idx, loss2d[0, 0]
